# Initial kernel scaffold; baseline (speedup 1.0000x reference)
import jax
import jax.numpy as jnp
from jax.experimental import pallas as pl


def kernel(x_categ, x_numer, image_condition_0, image_condition_1, params):
    raise NotImplementedError("write your pallas kernel here")


# speedup vs baseline: 801.5925x; 801.5925x over previous
import functools

import numpy as np
import jax
import jax.numpy as jnp
from jax.experimental import pallas as pl
from jax.experimental.pallas import tpu as pltpu

_DEPTH = 4
_CH = 64
_PE_DIM = 8
_WALK = 20
_NODE_DIM = 16
_DIM = 128
_H2 = 14
_W2 = 14
_NPG = _H2 * _W2
_GPAD = 200
_B = 4
_NB = _B * _GPAD


def _grid_edges_np(h, w):
    idx = np.arange(h * w).reshape(h, w)
    r0 = idx[:, :-1].ravel(); r1 = idx[:, 1:].ravel()
    d0 = idx[:-1, :].ravel(); d1 = idx[1:, :].ravel()
    src = np.concatenate([r0, r1, d0, d1])
    dst = np.concatenate([r1, r0, d1, d0])
    return np.stack([src, dst]).astype(np.int64)


def _rw_pe_np(edge, n, walk):
    A = np.zeros((n, n), dtype=np.float64)
    A[edge[0], edge[1]] = 1.0
    deg = A.sum(axis=1, keepdims=True)
    deg[deg == 0] = 1.0
    P = A / deg
    M = np.eye(n)
    pes = []
    for _ in range(walk):
        M = M @ P
        pes.append(np.diag(M))
    return np.stack(pes, axis=1).astype(np.float32)


_EDGE = _grid_edges_np(_H2, _W2)
_PE_RAW = _rw_pe_np(_EDGE, _NPG, _WALK)

_pe_m = _PE_RAW.astype(np.float64).mean(0)
_pe_v = _PE_RAW.astype(np.float64).var(0)
_PE_NORM = ((_PE_RAW - _pe_m) / np.sqrt(_pe_v + 1e-5)).astype(np.float32)
_PE_PAD = np.zeros((_NB, _WALK), np.float32)
for _b in range(_B):
    _PE_PAD[_b * _GPAD:_b * _GPAD + _NPG] = _PE_NORM

_A196 = np.zeros((_NPG, _NPG), np.float32)
_A196[_EDGE[1], _EDGE[0]] = 1.0
_A_PAD = np.zeros((_NB, _NB), np.float32)
for _b in range(_B):
    _A_PAD[_b * _GPAD:_b * _GPAD + _NPG, _b * _GPAD:_b * _GPAD + _NPG] = _A196

_PMAT = np.zeros((_B, _NB), np.float32)
for _b in range(_B):
    _PMAT[_b, _b * _GPAD:_b * _GPAD + _NPG] = 1.0

_JMAT = np.zeros((_H2, 224), np.float32)
for _r in range(224):
    _JMAT[_r // 16, _r] = 1.0
_KMAT = np.zeros((128, 16), np.float32)
for _c in range(128):
    _KMAT[_c, _c // 8] = 1.0

_E1 = np.zeros((_CH, _DIM), np.float32); _E1[:, :_CH] = np.eye(_CH)
_E2 = np.zeros((_CH, _DIM), np.float32); _E2[:, _CH:] = np.eye(_CH)


def _conv_body(img0, img1, wt0, wt1, jm, km, o0, o1):
    jmat = jm[...]
    kmat = km[...]
    for img, wt, out in ((img0, wt0, o0), (img1, wt1, o1)):
        x = img[0, 0]
        t = jnp.sum(x * wt[...], axis=0)
        pre = jax.lax.dot(jmat, t, preferred_element_type=jnp.float32)
        out[0, 0] = jax.lax.dot(pre, kmat, preferred_element_type=jnp.float32)


def _conv_call(im0, im1, wt0, wt1):
    return pl.pallas_call(
        _conv_body,
        grid=(_B, 14),
        in_specs=[
            pl.BlockSpec((1, 1, 16, 224, 128), lambda b, d: (b, 0, d, 0, 0)),
            pl.BlockSpec((1, 1, 16, 224, 128), lambda b, d: (b, 0, d, 0, 0)),
            pl.BlockSpec((16, 224, 128), lambda b, d: (0, 0, 0)),
            pl.BlockSpec((16, 224, 128), lambda b, d: (0, 0, 0)),
            pl.BlockSpec((_H2, 224), lambda b, d: (0, 0)),
            pl.BlockSpec((128, 16), lambda b, d: (0, 0)),
        ],
        out_specs=[
            pl.BlockSpec((1, 1, _H2, 16), lambda b, d: (b, d, 0, 0)),
            pl.BlockSpec((1, 1, _H2, 16), lambda b, d: (b, d, 0, 0)),
        ],
        out_shape=[jax.ShapeDtypeStruct((_B, 14, _H2, 16), jnp.float32)] * 2,
    )(im0, im1, wt0, wt1, jnp.asarray(_JMAT), jnp.asarray(_KMAT))


def _ln(x, g, b):
    m = jnp.mean(x, axis=-1, keepdims=True)
    d = x - m
    v = jnp.mean(d * d, axis=-1, keepdims=True)
    return d / jnp.sqrt(v + 1e-5) * g + b


def _gnn_body(x, peC, A, Pm, E1, E2, nw, pw, b64, e2, w1, b1, w2, b2, eps,
              lng, lnb, fw1, fb1, fw2, fb2, olng, olnb, ow, ob, out):
    f32 = jnp.float32
    Am = A[...]
    pe = peC[...]
    pooled = []
    for br in range(2):
        h = (jax.lax.dot(x[br], nw[br], preferred_element_type=f32)
             + jax.lax.dot(pe, pw[br], preferred_element_type=f32)
             + b64[br])
        for i in range(_DEPTH):
            R = jnp.maximum(h + e2[br], 0.0)
            agg = jax.lax.dot(Am, R, preferred_element_type=f32)
            z = (1.0 + eps[br, i]) * h + agg
            z = jnp.maximum(
                jax.lax.dot(z, w1[br, i], preferred_element_type=f32) + b1[br, i], 0.0)
            h = jax.lax.dot(z, w2[br, i], preferred_element_type=f32) + b2[br, i]
        pooled.append(jax.lax.dot(Pm[...], h, preferred_element_type=f32))
    whole = (jax.lax.dot(pooled[0], E1[...], preferred_element_type=f32)
             + jax.lax.dot(pooled[1], E2[...], preferred_element_type=f32))
    ff = _ln(whole, lng[...], lnb[...])
    ff = jax.nn.gelu(jax.lax.dot(ff, fw1[...], preferred_element_type=f32) + fb1[...])
    ff = jax.lax.dot(ff, fw2[...], preferred_element_type=f32) + fb2[...]
    xr = ff + whole
    xr = _ln(xr, olng[...], olnb[...])
    out[...] = jax.lax.dot(xr, ow[...], preferred_element_type=f32) + ob[...]


def _gnn_call(xs, ops):
    return pl.pallas_call(
        _gnn_body,
        out_shape=jax.ShapeDtypeStruct((_B, 1), jnp.float32),
    )(xs, jnp.asarray(_PE_PAD), jnp.asarray(_A_PAD), jnp.asarray(_PMAT),
      jnp.asarray(_E1), jnp.asarray(_E2), *ops)


def kernel(x_categ, x_numer, image_condition_0, image_condition_1, params):
    del x_categ, x_numer
    f32 = jnp.float32
    pm, pq = params['mri'], params['pet']

    wts = [jnp.tile(p['conv_w'][0, 0], (1, 14, 16)) for p in (pm, pq)]
    h0, h1 = _conv_call(image_condition_0, image_condition_1, wts[0], wts[1])

    def _pad_nodes(h):
        xg = h.reshape(_B, _NPG, _NODE_DIM)
        xg = jnp.pad(xg, ((0, 0), (0, _GPAD - _NPG), (0, 0)))
        return xg.reshape(_NB, _NODE_DIM)
    xs = jnp.stack([_pad_nodes(h0), _pad_nodes(h1)])

    def _fold(p):
        node_w = p['node_w'].astype(f32)
        nb = p['node_b'] + p['conv_b'][0] * jnp.sum(node_w, axis=0)
        pw_eff = p['pe_w'] * p['pe_bn_g'][:, None]
        pe_b_eff = p['pe_bn_b'] @ p['pe_w'] + p['pe_b']
        nw64 = jnp.pad(node_w, ((0, 0), (0, _PE_DIM)))
        pw64 = jnp.pad(pw_eff, ((0, 0), (_CH - _PE_DIM, 0)))
        b64 = jnp.concatenate([nb, pe_b_eff])[None, :]
        e = (p['edge_w'][0] + p['edge_b'])[None, :]
        return nw64, pw64, b64, e
    f0 = _fold(pm)
    f1 = _fold(pq)
    nw = jnp.stack([f0[0], f1[0]])
    pw = jnp.stack([f0[1], f1[1]])
    b64 = jnp.stack([f0[2], f1[2]])
    e2 = jnp.stack([f0[3], f1[3]])
    w1 = jnp.stack([pm['w1'], pq['w1']])
    b1 = jnp.stack([pm['b1'], pq['b1']])[:, :, None, :]
    w2 = jnp.stack([pm['w2'], pq['w2']])
    b2 = jnp.stack([pm['b2'], pq['b2']])[:, :, None, :]
    eps = jnp.stack([pm['eps'], pq['eps']])[:, :, None, None]

    ops = (nw, pw, b64, e2, w1, b1, w2, b2, eps,
           params['ff_ln_g'][None, :], params['ff_ln_b'][None, :],
           params['ff_w1'], params['ff_b1'][None, :],
           params['ff_w2'], params['ff_b2'][None, :],
           params['out_ln_g'][None, :], params['out_ln_b'][None, :],
           params['out_w'], params['out_b'][None, :])
    return _gnn_call(xs, ops)
